# Initial kernel scaffold; baseline (speedup 1.0000x reference)
import jax
import jax.numpy as jnp
from jax.experimental import pallas as pl


def kernel(x, table, W1, b1, W2, b2, W3, b3):
    raise NotImplementedError("write your pallas kernel here")


# speedup vs baseline: 14.0040x; 14.0040x over previous
import jax
import jax.numpy as jnp
from jax.experimental import pallas as pl
from jax.experimental.pallas import tpu as pltpu
from jax.experimental.pallas import tpu_sc as plsc

_MLP_TILE = 1024
_NC, _NS, _LANE = 2, 16, 128
_NW = _NC * _NS
_G = 16


def _sc_gather(table, idx_flat):
    n = idx_flat.shape[0]
    emb = table.shape[1]
    rows_per_worker = n // _NW
    assert n % (_NW * _LANE * _G) == 0
    idx_rows_pw = rows_per_worker // _LANE
    chunks = idx_rows_pw // _G
    idx2d = idx_flat.reshape(n // _LANE, _LANE)
    mesh = plsc.VectorSubcoreMesh(core_axis_name="core", subcore_axis_name="subcore")

    @pl.kernel(
        out_type=jax.ShapeDtypeStruct((n, emb), table.dtype),
        mesh=mesh,
        scratch_types=[
            pltpu.VMEM((_G, _LANE), jnp.int32),
            pltpu.VMEM((_G * _LANE, emb), table.dtype),
            pltpu.SemaphoreType.DMA,
        ],
        compiler_params=pltpu.CompilerParams(use_tc_tiling_on_sc=False),
    )
    def gather_kernel(tab_hbm, i_hbm, o_hbm, idx_v, rows_v, sem):
        wid = jax.lax.axis_index("subcore") * _NC + jax.lax.axis_index("core")
        row0 = wid * idx_rows_pw

        @pl.loop(0, chunks)
        def _(c):
            r = row0 + c * _G
            pltpu.sync_copy(i_hbm.at[pl.ds(r, _G)], idx_v)
            copies = [
                pltpu.async_copy(
                    tab_hbm.at[idx_v.at[j]],
                    rows_v.at[pl.ds(j * _LANE, _LANE)],
                    sem,
                )
                for j in range(_G)
            ]
            for cp in copies:
                cp.wait()
            pltpu.sync_copy(rows_v, o_hbm.at[pl.ds(r * _LANE, _G * _LANE)])

    return gather_kernel(table, idx2d)


def _mlp_body(h_ref, w1_ref, b1_ref, w2_ref, b2_ref, w3_ref, b3_ref, o_ref):
    h = h_ref[...]
    z = jnp.dot(h, w1_ref[...], preferred_element_type=jnp.float32) + b1_ref[...]
    z = jnp.maximum(z, 0.0)
    z = jnp.dot(z, w2_ref[...], preferred_element_type=jnp.float32) + b2_ref[...]
    z = jnp.maximum(z, 0.0)
    o_ref[...] = jnp.dot(z, w3_ref[...], preferred_element_type=jnp.float32) + b3_ref[...]


def _tc_mlp(h, W1, b1, W2, b2, W3, b3):
    batch, in_dim = h.shape
    h1, h2, ncls = W1.shape[1], W2.shape[1], W3.shape[1]
    tile = min(_MLP_TILE, batch)
    grid = (batch // tile,)
    full = lambda shape: pl.BlockSpec(shape, lambda i: (0,) * len(shape))
    return pl.pallas_call(
        _mlp_body,
        grid=grid,
        in_specs=[
            pl.BlockSpec((tile, in_dim), lambda i: (i, 0)),
            full((in_dim, h1)),
            full((1, h1)),
            full((h1, h2)),
            full((1, h2)),
            full((h2, ncls)),
            full((1, ncls)),
        ],
        out_specs=pl.BlockSpec((tile, ncls), lambda i: (i, 0)),
        out_shape=jax.ShapeDtypeStruct((batch, ncls), jnp.float32),
    )(h, W1, b1.reshape(1, h1), W2, b2.reshape(1, h2), W3, b3.reshape(1, ncls))


def kernel(x, table, W1, b1, W2, b2, W3, b3):
    batch, win = x.shape
    emb = table.shape[1]
    idx_flat = x.reshape(-1).astype(jnp.int32)
    embeds = _sc_gather(table, idx_flat)
    h = embeds.reshape(batch, win * emb)
    return _tc_mlp(h, W1, b1, W2, b2, W3, b3)


# speedup vs baseline: 14.8095x; 1.0575x over previous
import jax
import jax.numpy as jnp
from jax.experimental import pallas as pl
from jax.experimental.pallas import tpu as pltpu
from jax.experimental.pallas import tpu_sc as plsc

_MLP_TILE = 1024
_NC, _NS = 2, 16
_NW = _NC * _NS
_SB = 128


def _sc_gather(table, x):
    batch, win = x.shape
    emb = table.shape[1]
    kd = win // 4
    assert win % 4 == 0 and 4 * emb == 128
    npc = _SB * win
    assert npc % 128 == 0
    streams = npc // 128
    kblk = npc // kd
    b_per_w = batch // _NW
    chunks = b_per_w // _SB
    mesh = plsc.VectorSubcoreMesh(core_axis_name="core", subcore_axis_name="subcore")

    @pl.kernel(
        out_type=jax.ShapeDtypeStruct((kd * batch, 4 * emb), table.dtype),
        mesh=mesh,
        scratch_types=[
            pltpu.VMEM((_SB, win), jnp.int32),
            pltpu.VMEM((npc,), jnp.int32),
            pltpu.VMEM((npc, emb), table.dtype),
            pltpu.SemaphoreType.DMA,
        ],
        compiler_params=pltpu.CompilerParams(
            use_tc_tiling_on_sc=False, needs_layout_passes=False
        ),
    )
    def gather_kernel(tab_hbm, i_hbm, o_hbm, idx_v, idxp_v, rows_v, sem):
        wid = jax.lax.axis_index("subcore") * _NC + jax.lax.axis_index("core")
        b0 = wid * b_per_w
        lane = jax.lax.broadcasted_iota(jnp.int32, (16,), 0)

        @pl.loop(0, chunks)
        def _(c):
            b = b0 + c * _SB
            pltpu.sync_copy(i_hbm.at[pl.ds(b, _SB)], idx_v)
            for w in range(win):
                for u in range(_SB // 16):
                    rows = 16 * u + lane
                    cols = jnp.full((16,), w, jnp.int32)
                    vals = plsc.load_gather(idx_v, [rows, cols])
                    idxp_v[pl.ds(w * _SB + 16 * u, 16)] = vals
            copies = [
                pltpu.async_copy(
                    tab_hbm.at[idxp_v.at[pl.ds(j * 128, 128)]],
                    rows_v.at[pl.ds(j * 128, 128)],
                    sem,
                )
                for j in range(streams)
            ]
            for cp in copies:
                cp.wait()
            for w in range(win):
                k, c4 = divmod(w, 4)
                pltpu.sync_copy(
                    rows_v.at[pl.ds(w * _SB, _SB)],
                    o_hbm.at[pl.ds(k * batch + b, _SB), pl.ds(c4 * emb, emb)],
                )

    return gather_kernel(table, x)


def _mlp_body(h_ref, w1_ref, b1_ref, w2_ref, b2_ref, w3_ref, b3_ref, o_ref):
    kd = h_ref.shape[0]
    z = b1_ref[...] + jnp.dot(h_ref[0], w1_ref[0], preferred_element_type=jnp.float32)
    for k in range(1, kd):
        z = z + jnp.dot(h_ref[k], w1_ref[k], preferred_element_type=jnp.float32)
    z = jnp.maximum(z, 0.0)
    z = jnp.dot(z, w2_ref[...], preferred_element_type=jnp.float32) + b2_ref[...]
    z = jnp.maximum(z, 0.0)
    o_ref[...] = jnp.dot(z, w3_ref[...], preferred_element_type=jnp.float32) + b3_ref[...]


def _tc_mlp(h5, W1, b1, W2, b2, W3, b3):
    kd, batch, lane = h5.shape
    h1, h2, ncls = W1.shape[1], W2.shape[1], W3.shape[1]
    tile = min(_MLP_TILE, batch)
    grid = (batch // tile,)
    full = lambda shape: pl.BlockSpec(shape, lambda i: (0,) * len(shape))
    return pl.pallas_call(
        _mlp_body,
        grid=grid,
        in_specs=[
            pl.BlockSpec((kd, tile, lane), lambda i: (0, i, 0)),
            full((kd, lane, h1)),
            full((1, h1)),
            full((h1, h2)),
            full((1, h2)),
            full((h2, ncls)),
            full((1, ncls)),
        ],
        out_specs=pl.BlockSpec((tile, ncls), lambda i: (i, 0)),
        out_shape=jax.ShapeDtypeStruct((batch, ncls), jnp.float32),
    )(h5, W1.reshape(kd, lane, h1), b1.reshape(1, h1), W2, b2.reshape(1, h2),
      W3, b3.reshape(1, ncls))


def kernel(x, table, W1, b1, W2, b2, W3, b3):
    batch, win = x.shape
    xi = x.astype(jnp.int32)
    h = _sc_gather(table, xi)
    h5 = h.reshape(win // 4, batch, h.shape[-1])
    return _tc_mlp(h5, W1, b1, W2, b2, W3, b3)
